# baseline (speedup 1.0000x reference)
import functools

import jax
import jax.numpy as jnp
from jax import lax
from jax.experimental import pallas as pl
from jax.experimental.pallas import tpu as pltpu
from jax.experimental.pallas import tpu_sc as plsc

N = 10000
E = 320000
D = 128

NC = 2
NS = 16
NW = NC * NS

K = 128
CH = 80
EPT = CH * K
E2 = EPT * NW
XP = N + 8

DEGP = 10240
DROW = DEGP // NS

NP = 10112
RPT = NP // NS

_mesh = plsc.VectorSubcoreMesh(core_axis_name="c", subcore_axis_name="s")


@functools.partial(
    pl.kernel,
    out_type=jax.ShapeDtypeStruct((NC, DEGP), jnp.float32),
    mesh=_mesh,
    scratch_types=[
        pltpu.VMEM((CH, K), jnp.int32),
        pltpu.VMEM((K,), jnp.float32),
        pltpu.VMEM((DROW,), jnp.float32),
        pltpu.VMEM_SHARED((DEGP,), jnp.float32),
    ],
)
def _sc_deg(dst_hbm, out_hbm, didx, ones_v, zbuf, deg_s):
    c = lax.axis_index("c")
    s = lax.axis_index("s")
    wid = c * NS + s
    pltpu.sync_copy(dst_hbm.at[wid], didx)
    for j in range(K // 16):
        ones_v[pl.ds(j * 16, 16)] = jnp.ones((16,), jnp.float32)
    for j in range(DROW // 16):
        zbuf[pl.ds(j * 16, 16)] = jnp.zeros((16,), jnp.float32)
    pltpu.sync_copy(zbuf, deg_s.at[pl.ds(s * DROW, DROW)])
    plsc.subcore_barrier()

    def body(i, carry):
        pltpu.sync_copy(ones_v, deg_s.at[didx.at[i]], add=True)
        return carry

    lax.fori_loop(0, CH, body, 0)
    plsc.subcore_barrier()
    pltpu.sync_copy(deg_s.at[pl.ds(s * DROW, DROW)],
                    out_hbm.at[c, pl.ds(s * DROW, DROW)])


@functools.partial(
    pl.kernel,
    out_type=jax.ShapeDtypeStruct((NC, NP, D), jnp.float32),
    mesh=_mesh,
    scratch_types=[
        pltpu.VMEM((CH, K), jnp.int32),
        pltpu.VMEM((K,), jnp.int32),
        pltpu.VMEM((K,), jnp.int32),
        pltpu.VMEM((K, D), jnp.float32),
        pltpu.VMEM((K, D), jnp.float32),
        pltpu.VMEM_SHARED((NP, D), jnp.float32),
        pltpu.SemaphoreType.DMA,
        pltpu.SemaphoreType.DMA,
        pltpu.SemaphoreType.DMA,
        pltpu.SemaphoreType.DMA,
        pltpu.SemaphoreType.DMA,
        pltpu.SemaphoreType.DMA,
    ],
)
def _sc_msg(xs_hbm, src_hbm, dstf_hbm, zrow_hbm, out_hbm,
            sidx, d0, d1, rows0, rows1, acc_s, sG0, sG1, sS0, sS1, sD0, sD1):
    c = lax.axis_index("c")
    s = lax.axis_index("s")
    wid = c * NS + s
    ebase = wid * EPT
    pltpu.sync_copy(zrow_hbm, rows0)
    base = s * RPT
    for q in range(4):
        pltpu.sync_copy(rows0, acc_s.at[pl.ds(base + q * K, K)])
    pltpu.sync_copy(rows0.at[pl.ds(0, RPT - 4 * K)],
                    acc_s.at[pl.ds(base + 4 * K, RPT - 4 * K)])
    plsc.subcore_barrier()

    pltpu.sync_copy(src_hbm.at[wid], sidx)
    pltpu.sync_copy(dstf_hbm.at[pl.ds(ebase, K)], d0)
    pltpu.async_copy(xs_hbm.at[sidx.at[0]], rows0, sG0)

    def step(i, rows_c, rows_n, sG_c, sG_n, d_c, d_n, sD_c, sD_n, sS_c, sS_n):
        pltpu.make_async_copy(xs_hbm.at[pl.ds(0, K)], rows_c, sG_c).wait()

        @pl.when(i > 0)
        def _():
            pltpu.make_async_copy(rows_n, acc_s.at[pl.ds(0, K)], sS_n).wait()

        @pl.when(i + 1 < CH)
        def _():
            pltpu.async_copy(dstf_hbm.at[pl.ds(ebase + (i + 1) * K, K)],
                             d_n, sD_n)
            pltpu.async_copy(xs_hbm.at[sidx.at[i + 1]], rows_n, sG_n)

        @pl.when(i > 0)
        def _():
            pltpu.make_async_copy(dstf_hbm.at[pl.ds(ebase, K)], d_c,
                                  sD_c).wait()

        pltpu.async_copy(rows_c, acc_s.at[d_c], sS_c, add=True)

    def body(i, carry):
        @pl.when(lax.rem(i, 2) == 0)
        def _():
            step(i, rows0, rows1, sG0, sG1, d0, d1, sD0, sD1, sS0, sS1)

        @pl.when(lax.rem(i, 2) == 1)
        def _():
            step(i, rows1, rows0, sG1, sG0, d1, d0, sD1, sD0, sS1, sS0)

        return carry

    lax.fori_loop(0, CH, body, 0)
    pltpu.make_async_copy(rows1, acc_s.at[pl.ds(0, K)], sS1).wait()
    plsc.subcore_barrier()
    pltpu.sync_copy(acc_s.at[pl.ds(s * RPT, RPT)],
                    out_hbm.at[c, pl.ds(s * RPT, RPT)])


def _tc_mm_body(h_ref, w_ref, degt_ref, xs_ref):
    deg = degt_ref[:, 0:1] + degt_ref[:, 1:2] + 1.0
    dinv = lax.rsqrt(deg)
    x = jnp.dot(h_ref[:, :], w_ref[:, :], preferred_element_type=jnp.float32)
    xs_ref[:, :] = x * dinv


_tc_mm = pl.pallas_call(
    _tc_mm_body,
    out_shape=jax.ShapeDtypeStruct((N, D), jnp.float32),
)


def _tc_fin_body(accp_ref, xs_ref, degt_ref, b_ref, g_ref, be_ref, out_ref):
    deg = degt_ref[:, 0:1] + degt_ref[:, 1:2] + 1.0
    dinv = lax.rsqrt(deg)
    agg = accp_ref[0] + accp_ref[1] + xs_ref[:, :]
    o = agg * dinv + b_ref[:, :]
    mu = jnp.mean(o, axis=-1, keepdims=True)
    cen = o - mu
    var = jnp.mean(cen * cen, axis=-1, keepdims=True)
    y = cen * lax.rsqrt(var + 1e-5) * g_ref[:, :] + be_ref[:, :]
    out_ref[:, :] = jnp.tanh(y)


_tc_fin = pl.pallas_call(
    _tc_fin_body,
    out_shape=jax.ShapeDtypeStruct((N, D), jnp.float32),
)


def kernel(t, h, edge_index, batch_size, W, b, gamma, beta):
    if h.ndim == 1:
        h = h[None, :]
    pad = jnp.full((E2 - E,), N, dtype=edge_index.dtype)
    src = jnp.concatenate([edge_index[0], pad]).reshape(NW, CH, K)
    dstf = jnp.concatenate([edge_index[1], pad])
    dst = dstf.reshape(NW, CH, K)
    zrow = jnp.zeros((K, D), jnp.float32)

    degp = _sc_deg(dst)
    degt = jnp.stack([degp[0, :N], degp[1, :N]], axis=1)
    xs = _tc_mm(h, W, degt)
    xs_p = jnp.concatenate([xs, jnp.zeros((XP - N, D), jnp.float32)])
    accp = _sc_msg(xs_p, src, dstf, zrow)[:, :N]
    dh = _tc_fin(accp, xs, degt, b[None, :], gamma[None, :], beta[None, :])
    return (dh, jnp.zeros_like(edge_index), jnp.zeros_like(batch_size))


# speedup vs baseline: 2.4537x; 2.4537x over previous
import functools

import jax
import jax.numpy as jnp
from jax import lax
from jax.experimental import pallas as pl
from jax.experimental.pallas import tpu as pltpu
from jax.experimental.pallas import tpu_sc as plsc

N = 10000
E = 320000
D = 128

NC = 2
NS = 16
NW = NC * NS

K = 128
CH = 80
PH = CH // 2
EPT = CH * K
E2 = EPT * NW

DEGP = 10240
DROW = DEGP // NS

NP = 10112
RPT = NP // NS

_mesh = plsc.VectorSubcoreMesh(core_axis_name="c", subcore_axis_name="s")


@functools.partial(
    pl.kernel,
    out_type=jax.ShapeDtypeStruct((NC, DEGP), jnp.float32),
    mesh=_mesh,
    scratch_types=[
        pltpu.VMEM((CH, K), jnp.int32),
        pltpu.VMEM((K,), jnp.float32),
        pltpu.VMEM((DROW,), jnp.float32),
        pltpu.VMEM_SHARED((DEGP,), jnp.float32),
    ],
)
def _sc_deg(dst_hbm, out_hbm, didx, ones_v, zbuf, deg_s):
    c = lax.axis_index("c")
    s = lax.axis_index("s")
    wid = c * NS + s
    pltpu.sync_copy(dst_hbm.at[wid], didx)
    for j in range(K // 16):
        ones_v[pl.ds(j * 16, 16)] = jnp.ones((16,), jnp.float32)
    for j in range(DROW // 16):
        zbuf[pl.ds(j * 16, 16)] = jnp.zeros((16,), jnp.float32)
    pltpu.sync_copy(zbuf, deg_s.at[pl.ds(s * DROW, DROW)])
    plsc.subcore_barrier()

    def body(i, carry):
        pltpu.sync_copy(ones_v, deg_s.at[didx.at[i]], add=True)
        return carry

    lax.fori_loop(0, CH, body, 0)
    plsc.subcore_barrier()
    pltpu.sync_copy(deg_s.at[pl.ds(s * DROW, DROW)],
                    out_hbm.at[c, pl.ds(s * DROW, DROW)])


@functools.partial(
    pl.kernel,
    out_type=jax.ShapeDtypeStruct((NC, NP, D), jnp.float32),
    mesh=_mesh,
    scratch_types=[
        pltpu.VMEM((PH, K), jnp.int32),
        pltpu.VMEM((PH, K), jnp.int32),
        pltpu.VMEM((K, D), jnp.float32),
        pltpu.VMEM((K, D), jnp.float32),
        pltpu.VMEM_SHARED((NP, D), jnp.float32),
        pltpu.SemaphoreType.DMA,
        pltpu.SemaphoreType.DMA,
        pltpu.SemaphoreType.DMA,
        pltpu.SemaphoreType.DMA,
    ],
)
def _sc_msg(xs_hbm, src_hbm, dst_hbm, zrow_hbm, out_hbm,
            sidx, didx, rows0, rows1, acc_s, sG0, sG1, sS0, sS1):
    c = lax.axis_index("c")
    s = lax.axis_index("s")
    wid = c * NS + s
    base = s * RPT
    pltpu.sync_copy(zrow_hbm, rows0)
    sizes = [K, K, K, K, RPT - 4 * K]
    off = 0
    for sz in sizes:
        pltpu.sync_copy(rows0.at[pl.ds(0, sz)],
                        acc_s.at[pl.ds(base + off, sz)])
        off += sz
    plsc.subcore_barrier()

    def step(j, rows_c, rows_n, sG_c, sG_n, sS_c, sS_n):
        pltpu.make_async_copy(xs_hbm.at[pl.ds(0, K)], rows_c, sG_c).wait()

        @pl.when(j > 0)
        def _():
            pltpu.make_async_copy(rows_n, acc_s.at[pl.ds(0, K)], sS_n).wait()

        @pl.when(j + 1 < PH)
        def _():
            pltpu.async_copy(xs_hbm.at[sidx.at[j + 1]], rows_n, sG_n)

        pltpu.async_copy(rows_c, acc_s.at[didx.at[j]], sS_c, add=True)

    def body(j, carry):
        @pl.when(lax.rem(j, 2) == 0)
        def _():
            step(j, rows0, rows1, sG0, sG1, sS0, sS1)

        @pl.when(lax.rem(j, 2) == 1)
        def _():
            step(j, rows1, rows0, sG1, sG0, sS1, sS0)

        return carry

    for ph in range(CH // PH):
        pltpu.sync_copy(src_hbm.at[wid, pl.ds(ph * PH, PH)], sidx)
        pltpu.sync_copy(dst_hbm.at[wid, pl.ds(ph * PH, PH)], didx)
        pltpu.async_copy(xs_hbm.at[sidx.at[0]], rows0, sG0)
        lax.fori_loop(0, PH, body, 0)
        pltpu.make_async_copy(rows1, acc_s.at[pl.ds(0, K)], sS1).wait()

    plsc.subcore_barrier()
    pltpu.sync_copy(acc_s.at[pl.ds(base, RPT)],
                    out_hbm.at[c, pl.ds(base, RPT)])


def _tc_mm_body(h_ref, w_ref, degt_ref, xs_ref):
    deg = degt_ref[:, 0:1] + degt_ref[:, 1:2] + 1.0
    dinv = lax.rsqrt(deg)
    x = jnp.dot(h_ref[:, :], w_ref[:, :], preferred_element_type=jnp.float32)
    xs_ref[pl.ds(0, N), :] = x * dinv
    xs_ref[pl.ds(N, NP - N), :] = jnp.zeros((NP - N, D), jnp.float32)


_tc_mm = pl.pallas_call(
    _tc_mm_body,
    out_shape=jax.ShapeDtypeStruct((NP, D), jnp.float32),
)


def _tc_fin_body(accp_ref, xs_ref, degt_ref, b_ref, g_ref, be_ref, out_ref):
    deg = degt_ref[:, 0:1] + degt_ref[:, 1:2] + 1.0
    dinv = lax.rsqrt(deg)
    agg = (accp_ref[0, pl.ds(0, N), :] + accp_ref[1, pl.ds(0, N), :]
           + xs_ref[pl.ds(0, N), :])
    o = agg * dinv + b_ref[:, :]
    mu = jnp.mean(o, axis=-1, keepdims=True)
    cen = o - mu
    var = jnp.mean(cen * cen, axis=-1, keepdims=True)
    y = cen * lax.rsqrt(var + 1e-5) * g_ref[:, :] + be_ref[:, :]
    out_ref[:, :] = jnp.tanh(y)


_tc_fin = pl.pallas_call(
    _tc_fin_body,
    out_shape=jax.ShapeDtypeStruct((N, D), jnp.float32),
)


def kernel(t, h, edge_index, batch_size, W, b, gamma, beta):
    if h.ndim == 1:
        h = h[None, :]
    pad = N + (jnp.arange(E2 - E, dtype=edge_index.dtype) % (NP - N))
    src = jnp.concatenate([edge_index[0], pad]).reshape(NW, CH, K)
    dst = jnp.concatenate([edge_index[1], pad]).reshape(NW, CH, K)
    zrow = jnp.zeros((K, D), jnp.float32)

    degp = _sc_deg(dst)
    degt = jnp.stack([degp[0, :N], degp[1, :N]], axis=1)
    xs = _tc_mm(h, W, degt)
    accp = _sc_msg(xs, src, dst, zrow)
    dh = _tc_fin(accp, xs, degt, b[None, :], gamma[None, :], beta[None, :])
    return (dh, jnp.zeros_like(edge_index), jnp.zeros_like(batch_size))


# speedup vs baseline: 2.4539x; 1.0001x over previous
import functools

import jax
import jax.numpy as jnp
from jax import lax
from jax.experimental import pallas as pl
from jax.experimental.pallas import tpu as pltpu
from jax.experimental.pallas import tpu_sc as plsc

N = 10000
E = 320000
D = 128

NC = 2
NS = 16
NW = NC * NS

K = 128
CH = 80
PH = CH // 2
EPT = CH * K
E2 = EPT * NW

DEGP = 10240
DROW = DEGP // NS

NP = 10112
RPT = NP // NS

_mesh = plsc.VectorSubcoreMesh(core_axis_name="c", subcore_axis_name="s")


@functools.partial(
    pl.kernel,
    out_type=jax.ShapeDtypeStruct((NC, DEGP), jnp.float32),
    mesh=_mesh,
    scratch_types=[
        pltpu.VMEM((CH, K), jnp.int32),
        pltpu.VMEM((K,), jnp.float32),
        pltpu.VMEM((DROW,), jnp.float32),
        pltpu.VMEM_SHARED((DEGP,), jnp.float32),
    ],
)
def _sc_deg(dst_hbm, out_hbm, didx, ones_v, zbuf, deg_s):
    c = lax.axis_index("c")
    s = lax.axis_index("s")
    wid = c * NS + s
    pltpu.sync_copy(dst_hbm.at[wid], didx)
    for j in range(K // 16):
        ones_v[pl.ds(j * 16, 16)] = jnp.ones((16,), jnp.float32)
    for j in range(DROW // 16):
        zbuf[pl.ds(j * 16, 16)] = jnp.zeros((16,), jnp.float32)
    pltpu.sync_copy(zbuf, deg_s.at[pl.ds(s * DROW, DROW)])
    plsc.subcore_barrier()

    def body(i, carry):
        pltpu.sync_copy(ones_v, deg_s.at[didx.at[i]], add=True)
        return carry

    lax.fori_loop(0, CH, body, 0)
    plsc.subcore_barrier()
    pltpu.sync_copy(deg_s.at[pl.ds(s * DROW, DROW)],
                    out_hbm.at[c, pl.ds(s * DROW, DROW)])


@functools.partial(
    pl.kernel,
    out_type=jax.ShapeDtypeStruct((NC, NP, D), jnp.float32),
    mesh=_mesh,
    scratch_types=[
        pltpu.VMEM((PH, K), jnp.int32),
        pltpu.VMEM((PH, K), jnp.int32),
        pltpu.VMEM((K, D), jnp.float32),
        pltpu.VMEM((K, D), jnp.float32),
        pltpu.VMEM_SHARED((NP, D), jnp.float32),
        pltpu.SemaphoreType.DMA,
        pltpu.SemaphoreType.DMA,
        pltpu.SemaphoreType.DMA,
        pltpu.SemaphoreType.DMA,
    ],
)
def _sc_msg(xs_hbm, src_hbm, dst_hbm, zrow_hbm, out_hbm,
            sidx, didx, rows0, rows1, acc_s, sG0, sG1, sS0, sS1):
    c = lax.axis_index("c")
    s = lax.axis_index("s")
    wid = c * NS + s
    base = s * RPT
    pltpu.sync_copy(zrow_hbm, rows0)
    sizes = [K, K, K, K, RPT - 4 * K]
    off = 0
    for sz in sizes:
        pltpu.sync_copy(rows0.at[pl.ds(0, sz)],
                        acc_s.at[pl.ds(base + off, sz)])
        off += sz
    plsc.subcore_barrier()

    def wait_g(rows_b, sem):
        pltpu.make_async_copy(xs_hbm.at[pl.ds(0, K)], rows_b, sem).wait()

    def wait_s(rows_b, sem):
        pltpu.make_async_copy(rows_b, acc_s.at[pl.ds(0, K)], sem).wait()

    def body(m, carry):
        a = 2 * m
        wait_g(rows0, sG0)

        @pl.when(m > 0)
        def _():
            wait_s(rows1, sS1)

        pltpu.async_copy(xs_hbm.at[sidx.at[a + 1]], rows1, sG1)
        pltpu.async_copy(rows0, acc_s.at[didx.at[a]], sS0, add=True)
        wait_g(rows1, sG1)
        wait_s(rows0, sS0)

        @pl.when(a + 2 < PH)
        def _():
            pltpu.async_copy(xs_hbm.at[sidx.at[a + 2]], rows0, sG0)

        pltpu.async_copy(rows1, acc_s.at[didx.at[a + 1]], sS1, add=True)
        return carry

    for ph in range(CH // PH):
        pltpu.sync_copy(src_hbm.at[wid, pl.ds(ph * PH, PH)], sidx)
        pltpu.sync_copy(dst_hbm.at[wid, pl.ds(ph * PH, PH)], didx)
        pltpu.async_copy(xs_hbm.at[sidx.at[0]], rows0, sG0)
        lax.fori_loop(0, PH // 2, body, 0)
        wait_s(rows1, sS1)

    plsc.subcore_barrier()
    pltpu.sync_copy(acc_s.at[pl.ds(base, RPT)],
                    out_hbm.at[c, pl.ds(base, RPT)])


def _tc_mm_body(h_ref, w_ref, degt_ref, xs_ref):
    deg = degt_ref[:, 0:1] + degt_ref[:, 1:2] + 1.0
    dinv = lax.rsqrt(deg)
    x = jnp.dot(h_ref[:, :], w_ref[:, :], preferred_element_type=jnp.float32)
    xs_ref[pl.ds(0, N), :] = x * dinv
    xs_ref[pl.ds(N, NP - N), :] = jnp.zeros((NP - N, D), jnp.float32)


_tc_mm = pl.pallas_call(
    _tc_mm_body,
    out_shape=jax.ShapeDtypeStruct((NP, D), jnp.float32),
)


def _tc_fin_body(accp_ref, xs_ref, degt_ref, b_ref, g_ref, be_ref, out_ref):
    deg = degt_ref[:, 0:1] + degt_ref[:, 1:2] + 1.0
    dinv = lax.rsqrt(deg)
    agg = (accp_ref[0, pl.ds(0, N), :] + accp_ref[1, pl.ds(0, N), :]
           + xs_ref[pl.ds(0, N), :])
    o = agg * dinv + b_ref[:, :]
    mu = jnp.mean(o, axis=-1, keepdims=True)
    cen = o - mu
    var = jnp.mean(cen * cen, axis=-1, keepdims=True)
    y = cen * lax.rsqrt(var + 1e-5) * g_ref[:, :] + be_ref[:, :]
    out_ref[:, :] = jnp.tanh(y)


_tc_fin = pl.pallas_call(
    _tc_fin_body,
    out_shape=jax.ShapeDtypeStruct((N, D), jnp.float32),
)


def kernel(t, h, edge_index, batch_size, W, b, gamma, beta):
    if h.ndim == 1:
        h = h[None, :]
    pad = N + (jnp.arange(E2 - E, dtype=edge_index.dtype) % (NP - N))
    src = jnp.concatenate([edge_index[0], pad]).reshape(NW, CH, K)
    dst = jnp.concatenate([edge_index[1], pad]).reshape(NW, CH, K)
    zrow = jnp.zeros((K, D), jnp.float32)

    degp = _sc_deg(dst)
    degt = jnp.stack([degp[0, :N], degp[1, :N]], axis=1)
    xs = _tc_mm(h, W, degt)
    accp = _sc_msg(xs, src, dst, zrow)
    dh = _tc_fin(accp, xs, degt, b[None, :], gamma[None, :], beta[None, :])
    return (dh, jnp.zeros_like(edge_index), jnp.zeros_like(batch_size))


# speedup vs baseline: 2.5332x; 1.0323x over previous
import functools

import jax
import jax.numpy as jnp
from jax import lax
from jax.experimental import pallas as pl
from jax.experimental.pallas import tpu as pltpu
from jax.experimental.pallas import tpu_sc as plsc

N = 10000
E = 320000
D = 128

NC = 2
NS = 16
NW = NC * NS

K = 128
CH = 80
PH = CH // 2
EPT = CH * K
E2 = EPT * NW

DEGP = 10240
DROW = DEGP // NS

NP = 10112
RPT = NP // NS

_mesh = plsc.VectorSubcoreMesh(core_axis_name="c", subcore_axis_name="s")


@functools.partial(
    pl.kernel,
    out_type=jax.ShapeDtypeStruct((NC, DEGP), jnp.float32),
    mesh=_mesh,
    scratch_types=[
        pltpu.VMEM((CH, K), jnp.int32),
        pltpu.VMEM((K,), jnp.float32),
        pltpu.VMEM((DROW,), jnp.float32),
        pltpu.VMEM_SHARED((DEGP,), jnp.float32),
    ],
)
def _sc_deg(dst_hbm, out_hbm, didx, ones_v, zbuf, deg_s):
    c = lax.axis_index("c")
    s = lax.axis_index("s")
    wid = c * NS + s
    pltpu.sync_copy(dst_hbm.at[wid], didx)
    for j in range(K // 16):
        ones_v[pl.ds(j * 16, 16)] = jnp.ones((16,), jnp.float32)
    for j in range(DROW // 16):
        zbuf[pl.ds(j * 16, 16)] = jnp.zeros((16,), jnp.float32)
    pltpu.sync_copy(zbuf, deg_s.at[pl.ds(s * DROW, DROW)])
    plsc.subcore_barrier()

    def body(i, carry):
        pltpu.sync_copy(ones_v, deg_s.at[didx.at[i]], add=True)
        return carry

    lax.fori_loop(0, CH, body, 0)
    plsc.subcore_barrier()
    pltpu.sync_copy(deg_s.at[pl.ds(s * DROW, DROW)],
                    out_hbm.at[c, pl.ds(s * DROW, DROW)])


@functools.partial(
    pl.kernel,
    out_type=jax.ShapeDtypeStruct((NC, NP, D), jnp.float32),
    mesh=_mesh,
    scratch_types=[
        pltpu.VMEM((PH, K), jnp.int32),
        pltpu.VMEM((PH, K), jnp.int32),
        pltpu.VMEM((K, D), jnp.float32),
        pltpu.VMEM((K, D), jnp.float32),
        pltpu.VMEM_SHARED((NP, D), jnp.float32),
        pltpu.SemaphoreType.DMA,
        pltpu.SemaphoreType.DMA,
        pltpu.SemaphoreType.DMA,
        pltpu.SemaphoreType.DMA,
    ],
)
def _sc_msg(xs_hbm, src_hbm, dst_hbm, zrow_hbm, out_hbm,
            sidx, didx, rows0, rows1, acc_s, sG0, sG1, sS0, sS1):
    c = lax.axis_index("c")
    s = lax.axis_index("s")
    wid = c * NS + s
    base = s * RPT
    pltpu.sync_copy(zrow_hbm, rows0)
    sizes = [K, K, K, K, RPT - 4 * K]
    off = 0
    for sz in sizes:
        pltpu.sync_copy(rows0.at[pl.ds(0, sz)],
                        acc_s.at[pl.ds(base + off, sz)])
        off += sz
    plsc.subcore_barrier()

    def wait_g(rows_b, sem):
        pltpu.make_async_copy(xs_hbm.at[pl.ds(0, K)], rows_b, sem).wait()

    def wait_s(rows_b, sem):
        pltpu.make_async_copy(rows_b, acc_s.at[pl.ds(0, K)], sem).wait()

    def body(m, carry):
        a = 2 * m
        wait_g(rows0, sG0)

        @pl.when(m > 0)
        def _():
            wait_s(rows1, sS1)

        pltpu.async_copy(xs_hbm.at[sidx.at[a + 1]], rows1, sG1)
        pltpu.async_copy(rows0, acc_s.at[didx.at[a]], sS0, add=True)
        wait_g(rows1, sG1)
        wait_s(rows0, sS0)

        @pl.when(a + 2 < PH)
        def _():
            pltpu.async_copy(xs_hbm.at[sidx.at[a + 2]], rows0, sG0)

        pltpu.async_copy(rows1, acc_s.at[didx.at[a + 1]], sS1, add=True)
        return carry

    for ph in range(CH // PH):
        pltpu.sync_copy(src_hbm.at[wid, pl.ds(ph * PH, PH)], sidx)
        pltpu.sync_copy(dst_hbm.at[wid, pl.ds(ph * PH, PH)], didx)
        pltpu.async_copy(xs_hbm.at[sidx.at[0]], rows0, sG0)
        lax.fori_loop(0, PH // 2, body, 0)
        wait_s(rows1, sS1)

    plsc.subcore_barrier()
    pltpu.sync_copy(acc_s.at[pl.ds(base, RPT)],
                    out_hbm.at[c, pl.ds(base, RPT)])


def _dinv_col(degp_ref):
    dp = degp_ref[:, pl.ds(0, N)]
    dsum = dp[0:1, :] + dp[1:2, :] + 1.0
    return lax.rsqrt(jnp.transpose(dsum, (1, 0)))


def _tc_mm_body(h_ref, w_ref, x_ref):
    x_ref[:, :] = jnp.dot(h_ref[:, :], w_ref[:, :],
                          preferred_element_type=jnp.float32)


_tc_mm = pl.pallas_call(
    _tc_mm_body,
    out_shape=jax.ShapeDtypeStruct((N, D), jnp.float32),
)


def _tc_scale_body(x_ref, degp_ref, xs_ref):
    xs_ref[pl.ds(0, N), :] = x_ref[:, :] * _dinv_col(degp_ref)
    xs_ref[pl.ds(N, NP - N), :] = jnp.zeros((NP - N, D), jnp.float32)


_tc_scale = pl.pallas_call(
    _tc_scale_body,
    out_shape=jax.ShapeDtypeStruct((NP, D), jnp.float32),
)


def _tc_fin_body(accp_ref, xs_ref, degp_ref, b_ref, g_ref, be_ref, out_ref):
    dinv = _dinv_col(degp_ref)
    agg = (accp_ref[0, pl.ds(0, N), :] + accp_ref[1, pl.ds(0, N), :]
           + xs_ref[pl.ds(0, N), :])
    o = agg * dinv + b_ref[:, :]
    mu = jnp.mean(o, axis=-1, keepdims=True)
    cen = o - mu
    var = jnp.mean(cen * cen, axis=-1, keepdims=True)
    y = cen * lax.rsqrt(var + 1e-5) * g_ref[:, :] + be_ref[:, :]
    out_ref[:, :] = jnp.tanh(y)


_tc_fin = pl.pallas_call(
    _tc_fin_body,
    out_shape=jax.ShapeDtypeStruct((N, D), jnp.float32),
)


def kernel(t, h, edge_index, batch_size, W, b, gamma, beta):
    if h.ndim == 1:
        h = h[None, :]
    pad = N + (jnp.arange(E2 - E, dtype=edge_index.dtype) % (NP - N))
    src = jnp.concatenate([edge_index[0], pad]).reshape(NW, CH, K)
    dst = jnp.concatenate([edge_index[1], pad]).reshape(NW, CH, K)
    zrow = jnp.zeros((K, D), jnp.float32)

    degp = _sc_deg(dst)
    x = _tc_mm(h, W)
    xs = _tc_scale(x, degp)
    accp = _sc_msg(xs, src, dst, zrow)
    dh = _tc_fin(accp, xs, degp, b[None, :], gamma[None, :], beta[None, :])
    return (dh, jnp.zeros_like(edge_index), jnp.zeros_like(batch_size))


# speedup vs baseline: 2.5358x; 1.0011x over previous
import functools

import jax
import jax.numpy as jnp
from jax import lax
from jax.experimental import pallas as pl
from jax.experimental.pallas import tpu as pltpu
from jax.experimental.pallas import tpu_sc as plsc

N = 10000
E = 320000
D = 128

NC = 2
NS = 16
NW = NC * NS

K = 128
CH = 80
PH = CH // 2
EPT = CH * K
E2 = EPT * NW

DEGP = 10240
DROW = DEGP // NS

NP = 10112
RPT = NP // NS

_mesh = plsc.VectorSubcoreMesh(core_axis_name="c", subcore_axis_name="s")


@functools.partial(
    pl.kernel,
    out_type=jax.ShapeDtypeStruct((NC, DEGP), jnp.float32),
    mesh=_mesh,
    scratch_types=[
        pltpu.VMEM((CH, K), jnp.int32),
        pltpu.VMEM((K,), jnp.float32),
        pltpu.VMEM((DROW,), jnp.float32),
        pltpu.VMEM_SHARED((DEGP,), jnp.float32),
    ],
)
def _sc_deg(dst_hbm, out_hbm, didx, ones_v, zbuf, deg_s):
    c = lax.axis_index("c")
    s = lax.axis_index("s")
    wid = c * NS + s
    pltpu.sync_copy(dst_hbm.at[wid], didx)
    for j in range(K // 16):
        ones_v[pl.ds(j * 16, 16)] = jnp.ones((16,), jnp.float32)
    for j in range(DROW // 16):
        zbuf[pl.ds(j * 16, 16)] = jnp.zeros((16,), jnp.float32)
    pltpu.sync_copy(zbuf, deg_s.at[pl.ds(s * DROW, DROW)])
    plsc.subcore_barrier()

    def body(i, carry):
        pltpu.sync_copy(ones_v, deg_s.at[didx.at[i]], add=True)
        return carry

    lax.fori_loop(0, CH, body, 0)
    plsc.subcore_barrier()
    pltpu.sync_copy(deg_s.at[pl.ds(s * DROW, DROW)],
                    out_hbm.at[c, pl.ds(s * DROW, DROW)])


@functools.partial(
    pl.kernel,
    out_type=jax.ShapeDtypeStruct((NC, NP, D), jnp.float32),
    mesh=_mesh,
    scratch_types=[
        pltpu.VMEM((PH, K), jnp.int32),
        pltpu.VMEM((PH, K), jnp.int32),
        pltpu.VMEM((K, D), jnp.float32),
        pltpu.VMEM((K, D), jnp.float32),
        pltpu.VMEM_SHARED((NP, D), jnp.float32),
        pltpu.SemaphoreType.DMA,
        pltpu.SemaphoreType.DMA,
        pltpu.SemaphoreType.DMA,
        pltpu.SemaphoreType.DMA,
    ],
)
def _sc_msg(xs_hbm, src_hbm, dst_hbm, zrow_hbm, out_hbm,
            sidx, didx, rows0, rows1, acc_s, sG0, sG1, sS0, sS1):
    c = lax.axis_index("c")
    s = lax.axis_index("s")
    wid = c * NS + s
    base = s * RPT
    pltpu.sync_copy(zrow_hbm, rows0)
    sizes = [K, K, K, K, RPT - 4 * K]
    off = 0
    for sz in sizes:
        pltpu.sync_copy(rows0.at[pl.ds(0, sz)],
                        acc_s.at[pl.ds(base + off, sz)])
        off += sz
    plsc.subcore_barrier()

    HK = K // 2

    def start_g(j, rows_b, sem):
        pltpu.async_copy(xs_hbm.at[sidx.at[j, pl.ds(0, HK)]],
                         rows_b.at[pl.ds(0, HK)], sem)
        pltpu.async_copy(xs_hbm.at[sidx.at[j, pl.ds(HK, HK)]],
                         rows_b.at[pl.ds(HK, HK)], sem)

    def wait_g(rows_b, sem):
        for _ in range(2):
            pltpu.make_async_copy(xs_hbm.at[pl.ds(0, HK)],
                                  rows_b.at[pl.ds(0, HK)], sem).wait()

    def wait_s(rows_b, sem):
        pltpu.make_async_copy(rows_b, acc_s.at[pl.ds(0, K)], sem).wait()

    def body(m, carry):
        a = 2 * m
        wait_g(rows0, sG0)

        @pl.when(m > 0)
        def _():
            wait_s(rows1, sS1)

        start_g(a + 1, rows1, sG1)
        pltpu.async_copy(rows0, acc_s.at[didx.at[a]], sS0, add=True)
        wait_g(rows1, sG1)
        wait_s(rows0, sS0)

        @pl.when(a + 2 < PH)
        def _():
            start_g(a + 2, rows0, sG0)

        pltpu.async_copy(rows1, acc_s.at[didx.at[a + 1]], sS1, add=True)
        return carry

    for ph in range(CH // PH):
        pltpu.sync_copy(src_hbm.at[wid, pl.ds(ph * PH, PH)], sidx)
        pltpu.sync_copy(dst_hbm.at[wid, pl.ds(ph * PH, PH)], didx)
        start_g(0, rows0, sG0)
        lax.fori_loop(0, PH // 2, body, 0)
        wait_s(rows1, sS1)

    plsc.subcore_barrier()
    pltpu.sync_copy(acc_s.at[pl.ds(base, RPT)],
                    out_hbm.at[c, pl.ds(base, RPT)])


def _dinv_col(degp_ref):
    dp = degp_ref[:, pl.ds(0, N)]
    dsum = dp[0:1, :] + dp[1:2, :] + 1.0
    return lax.rsqrt(jnp.transpose(dsum, (1, 0)))


def _tc_mm_body(h_ref, w_ref, x_ref):
    x_ref[:, :] = jnp.dot(h_ref[:, :], w_ref[:, :],
                          preferred_element_type=jnp.float32)


_tc_mm = pl.pallas_call(
    _tc_mm_body,
    out_shape=jax.ShapeDtypeStruct((N, D), jnp.float32),
)


def _tc_scale_body(x_ref, degp_ref, xs_ref):
    xs_ref[pl.ds(0, N), :] = x_ref[:, :] * _dinv_col(degp_ref)
    xs_ref[pl.ds(N, NP - N), :] = jnp.zeros((NP - N, D), jnp.float32)


_tc_scale = pl.pallas_call(
    _tc_scale_body,
    out_shape=jax.ShapeDtypeStruct((NP, D), jnp.float32),
)


def _tc_fin_body(accp_ref, xs_ref, degp_ref, b_ref, g_ref, be_ref, out_ref):
    dinv = _dinv_col(degp_ref)
    agg = (accp_ref[0, pl.ds(0, N), :] + accp_ref[1, pl.ds(0, N), :]
           + xs_ref[pl.ds(0, N), :])
    o = agg * dinv + b_ref[:, :]
    mu = jnp.mean(o, axis=-1, keepdims=True)
    cen = o - mu
    var = jnp.mean(cen * cen, axis=-1, keepdims=True)
    y = cen * lax.rsqrt(var + 1e-5) * g_ref[:, :] + be_ref[:, :]
    out_ref[:, :] = jnp.tanh(y)


_tc_fin = pl.pallas_call(
    _tc_fin_body,
    out_shape=jax.ShapeDtypeStruct((N, D), jnp.float32),
)


def kernel(t, h, edge_index, batch_size, W, b, gamma, beta):
    if h.ndim == 1:
        h = h[None, :]
    pad = N + (jnp.arange(E2 - E, dtype=edge_index.dtype) % (NP - N))
    src = jnp.concatenate([edge_index[0], pad]).reshape(NW, CH, K)
    dst = jnp.concatenate([edge_index[1], pad]).reshape(NW, CH, K)
    zrow = jnp.zeros((K, D), jnp.float32)

    degp = _sc_deg(dst)
    x = _tc_mm(h, W)
    xs = _tc_scale(x, degp)
    accp = _sc_msg(xs, src, dst, zrow)
    dh = _tc_fin(accp, xs, degp, b[None, :], gamma[None, :], beta[None, :])
    return (dh, jnp.zeros_like(edge_index), jnp.zeros_like(batch_size))
